# Initial kernel scaffold; baseline (speedup 1.0000x reference)
import jax
import jax.numpy as jnp
from jax.experimental import pallas as pl


def kernel(x, table):
    raise NotImplementedError("write your pallas kernel here")


# speedup vs baseline: 3.2012x; 3.2012x over previous
import functools

import jax
import jax.numpy as jnp
from jax import lax
from jax.experimental import pallas as pl
from jax.experimental.pallas import tpu as pltpu
from jax.experimental.pallas import tpu_sc as plsc

EMB = 128
CHUNK = 400


def _sc_embedding_lookup(x_flat, table):
    n = x_flat.shape[0]
    info = plsc.get_sparse_core_info()
    nw = info.num_cores * info.num_subcores
    per_w = n // nw
    n_iters = per_w // CHUNK
    assert per_w % CHUNK == 0 and n % nw == 0

    mesh = plsc.VectorSubcoreMesh(core_axis_name="c", subcore_axis_name="s")

    @functools.partial(
        pl.kernel,
        mesh=mesh,
        out_type=jax.ShapeDtypeStruct((n, EMB), jnp.float32),
        scratch_types=[
            pltpu.VMEM((CHUNK,), jnp.int32),
            pltpu.VMEM((CHUNK, EMB), jnp.float32),
            pltpu.SemaphoreType.DMA,
        ],
    )
    def k(x_hbm, table_hbm, out_hbm, idx_v, rows_v, sem):
        wid = lax.axis_index("s") * info.num_cores + lax.axis_index("c")
        base = wid * per_w

        def body(i, carry):
            start = base + i * CHUNK
            pltpu.sync_copy(x_hbm.at[pl.ds(start, CHUNK)], idx_v)
            pltpu.async_copy(table_hbm.at[idx_v], rows_v, sem).wait()
            pltpu.sync_copy(rows_v, out_hbm.at[pl.ds(start, CHUNK)])
            return carry

        lax.fori_loop(0, n_iters, body, 0)

    return k(x_flat, table)


def kernel(x, table):
    b, h = x.shape
    out = _sc_embedding_lookup(x.reshape(b * h), table)
    return out.reshape(b, h, EMB)


# speedup vs baseline: 3.2255x; 1.0076x over previous
import functools

import jax
import jax.numpy as jnp
from jax import lax
from jax.experimental import pallas as pl
from jax.experimental.pallas import tpu as pltpu
from jax.experimental.pallas import tpu_sc as plsc

EMB = 128
CHUNK = 400
IB = 16


def _sc_embedding_lookup(x_flat, table):
    n = x_flat.shape[0]
    info = plsc.get_sparse_core_info()
    nw = info.num_cores * info.num_subcores
    per_w = n // nw
    n_iters = per_w // CHUNK
    assert per_w % CHUNK == 0 and n % nw == 0 and n_iters % IB == 0

    mesh = plsc.VectorSubcoreMesh(core_axis_name="c", subcore_axis_name="s")

    @functools.partial(
        pl.kernel,
        mesh=mesh,
        out_type=jax.ShapeDtypeStruct((n, EMB), jnp.float32),
        scratch_types=[
            pltpu.VMEM((IB * CHUNK,), jnp.int32),
            pltpu.VMEM((2, CHUNK, EMB), jnp.float32),
            pltpu.SemaphoreType.DMA,
            pltpu.SemaphoreType.DMA,
        ],
    )
    def k(x_hbm, table_hbm, out_hbm, idx_v, rows_v, sem_g, sem_s):
        wid = lax.axis_index("s") * info.num_cores + lax.axis_index("c")
        base = wid * per_w

        def body(i, carry):
            slot = i % 2
            j = i % IB
            start = base + i * CHUNK

            @pl.when(j == 0)
            def _():
                pltpu.sync_copy(x_hbm.at[pl.ds(start, IB * CHUNK)], idx_v)

            pltpu.async_copy(
                table_hbm.at[idx_v.at[pl.ds(j * CHUNK, CHUNK)]],
                rows_v.at[slot],
                sem_g,
            ).wait()

            @pl.when(i >= 1)
            def _():
                pltpu.make_async_copy(
                    rows_v.at[1 - slot],
                    out_hbm.at[pl.ds(base, CHUNK)],
                    sem_s,
                ).wait()

            pltpu.async_copy(
                rows_v.at[slot], out_hbm.at[pl.ds(start, CHUNK)], sem_s
            )
            return carry

        lax.fori_loop(0, n_iters, body, 0)
        pltpu.make_async_copy(
            rows_v.at[(n_iters - 1) % 2], out_hbm.at[pl.ds(base, CHUNK)], sem_s
        ).wait()

    return k(x_flat, table)


def kernel(x, table):
    b, h = x.shape
    out = _sc_embedding_lookup(x.reshape(b * h), table)
    return out.reshape(b, h, EMB)
